# baseline (speedup 1.0000x reference)
import functools

import jax
import jax.numpy as jnp
from jax import lax
from jax.experimental import pallas as pl
from jax.experimental.pallas import tpu as pltpu
from jax.experimental.pallas import tpu_sc as plsc

_LANES = 16


def kernel(pos, idx):
    batch, atm, dim = pos.shape
    n_idx = idx.shape[0]
    rows = batch * n_idx
    owords = rows * dim

    info = plsc.get_sparse_core_info()
    nc, ns = info.num_cores, info.num_subcores
    nw = nc * ns
    per_w = rows // nw
    opw = per_w * dim
    steps = opw // _LANES

    nwords = batch * atm * dim
    nwin = nwords // _LANES
    table = pos.reshape(nwin, _LANES)

    gidx = (
        jnp.arange(batch, dtype=jnp.int32)[:, None] * atm
        + idx.astype(jnp.int32)[None, :]
    ).reshape(rows)
    start = gidx * dim
    win0 = start >> 4
    win1 = jnp.minimum(win0 + 1, nwin - 1)
    win_idx = jnp.stack([win0, win1], axis=1).reshape(rows * 2)

    k = jnp.arange(owords, dtype=jnp.int32)
    i = k // dim
    addr = (i % per_w) * (2 * _LANES) + (start[i] & (_LANES - 1)) + k % dim
    rix = addr >> 4
    cix = addr & (_LANES - 1)

    mesh = plsc.VectorSubcoreMesh(core_axis_name="c", subcore_axis_name="s")

    @functools.partial(
        pl.kernel,
        mesh=mesh,
        out_type=jax.ShapeDtypeStruct((owords,), jnp.float32),
        compiler_params=pltpu.CompilerParams(
            use_tc_tiling_on_sc=False, needs_layout_passes=False),
        scratch_types=[
            pltpu.VMEM((per_w * 2,), jnp.int32),
            pltpu.VMEM((per_w * 2, _LANES), jnp.float32),
            pltpu.VMEM((opw,), jnp.int32),
            pltpu.VMEM((opw,), jnp.int32),
            pltpu.VMEM((opw,), jnp.float32),
            pltpu.SemaphoreType.DMA,
        ],
    )
    def gather_rows(tab_hbm, win_hbm, rix_hbm, cix_hbm, out_hbm,
                    widx_v, win_v, rix_v, cix_v, out_v, sem):
        wid = lax.axis_index("s") * nc + lax.axis_index("c")
        pltpu.sync_copy(win_hbm.at[pl.ds(wid * per_w * 2, per_w * 2)], widx_v)
        pltpu.sync_copy(rix_hbm.at[pl.ds(wid * opw, opw)], rix_v)
        pltpu.sync_copy(cix_hbm.at[pl.ds(wid * opw, opw)], cix_v)
        pltpu.async_copy(tab_hbm.at[widx_v], win_v, sem).wait()
        for t in range(steps):
            rvec = rix_v[pl.ds(t * _LANES, _LANES)]
            cvec = cix_v[pl.ds(t * _LANES, _LANES)]
            out_v[pl.ds(t * _LANES, _LANES)] = plsc.load_gather(
                win_v, [rvec, cvec])
        pltpu.sync_copy(out_v, out_hbm.at[pl.ds(wid * opw, opw)])

    out = gather_rows(table, win_idx, rix, cix)
    return out.reshape(batch, n_idx * dim)


# speedup vs baseline: 718.2298x; 718.2298x over previous
import functools

import jax
import jax.numpy as jnp
from jax import lax
from jax.experimental import pallas as pl
from jax.experimental.pallas import tpu as pltpu
from jax.experimental.pallas import tpu_sc as plsc

_LANES = 16
_TILE = 128


def kernel(pos, idx):
    batch, atm, dim = pos.shape
    n_idx = idx.shape[0]
    nrow = dim * batch
    pos2 = pos.transpose(2, 0, 1).reshape(nrow, atm)

    idx32 = idx.astype(jnp.int32)
    w0 = (idx32 // _TILE) * _TILE
    l128 = idx32 % _TILE

    info = plsc.get_sparse_core_info()
    nc, ns = info.num_cores, info.num_subcores
    nw = nc * ns
    jpw = n_idx // nw
    opw = jpw * nrow
    steps = nrow // _LANES

    params = jnp.zeros((nw, _LANES), jnp.int32)
    params = params.at[:, :jpw].set(w0.reshape(nw, jpw))
    params = params.at[:, jpw:2 * jpw].set(l128.reshape(nw, jpw))
    params = params.reshape(nw * _LANES)

    mesh = plsc.VectorSubcoreMesh(core_axis_name="c", subcore_axis_name="s")

    @functools.partial(
        pl.kernel,
        mesh=mesh,
        out_type=jax.ShapeDtypeStruct((n_idx * nrow,), jnp.float32),
        compiler_params=pltpu.CompilerParams(needs_layout_passes=False),
        scratch_types=[
            pltpu.VMEM((nw * _LANES,), jnp.int32),
            pltpu.VMEM((nrow, _TILE), jnp.float32),
            pltpu.VMEM((opw,), jnp.float32),
            pltpu.SemaphoreType.DMA,
        ],
    )
    def gather_cols(pos_hbm, par_hbm, out_hbm, par_v, win_v, out_v, sem):
        wid = lax.axis_index("s") * nc + lax.axis_index("c")
        pltpu.sync_copy(par_hbm, par_v)
        pvec = par_v[pl.ds(wid * _LANES, _LANES)]
        riota = lax.broadcasted_iota(jnp.int32, (_LANES,), 0)
        for p in range(jpw):
            c0 = pl.multiple_of(
                jnp.max(jnp.where(riota == p, pvec, 0)), _TILE)
            lane = jnp.max(jnp.where(riota == jpw + p, pvec, 0))
            cvec = jnp.full((_LANES,), lane, jnp.int32)
            pltpu.sync_copy(pos_hbm.at[:, pl.ds(c0, _TILE)], win_v)
            for t in range(steps):
                out_v[pl.ds(p * nrow + t * _LANES, _LANES)] = plsc.load_gather(
                    win_v, [riota + t * _LANES, cvec])
        pltpu.sync_copy(out_v, out_hbm.at[pl.ds(wid * opw, opw)])

    out = gather_cols(pos2, params)
    return (out.reshape(n_idx, dim, batch)
            .transpose(2, 0, 1)
            .reshape(batch, n_idx * dim))


# speedup vs baseline: 781.7634x; 1.0885x over previous
import functools

import jax
import jax.numpy as jnp
from jax import lax
from jax.experimental import pallas as pl
from jax.experimental.pallas import tpu as pltpu
from jax.experimental.pallas import tpu_sc as plsc

_LANES = 16
_TILE = 128


def kernel(pos, idx):
    batch, atm, dim = pos.shape
    n_idx = idx.shape[0]
    nrow = dim * batch
    pos2 = pos.transpose(2, 0, 1).reshape(nrow, atm)
    idx32 = idx.astype(jnp.int32)

    info = plsc.get_sparse_core_info()
    nc, ns = info.num_cores, info.num_subcores
    nw = nc * ns
    jpw = n_idx // nw
    opw = jpw * nrow
    steps = nrow // _LANES

    mesh = plsc.VectorSubcoreMesh(core_axis_name="c", subcore_axis_name="s")

    @functools.partial(
        pl.kernel,
        mesh=mesh,
        out_type=jax.ShapeDtypeStruct((n_idx * nrow,), jnp.float32),
        compiler_params=pltpu.CompilerParams(needs_layout_passes=False),
        scratch_types=[
            pltpu.VMEM((n_idx,), jnp.int32),
            [pltpu.VMEM((nrow, _TILE), jnp.float32)] * jpw,
            pltpu.VMEM((opw,), jnp.float32),
            [pltpu.SemaphoreType.DMA] * jpw,
        ],
    )
    def gather_cols(pos_hbm, idx_hbm, out_hbm, idx_v, wins, out_v, sems):
        wid = lax.axis_index("s") * nc + lax.axis_index("c")
        pltpu.sync_copy(idx_hbm, idx_v)
        riota = lax.broadcasted_iota(jnp.int32, (_LANES,), 0)
        lanes = []
        copies = []
        for p in range(jpw):
            j = wid * jpw + p
            chunk = pl.multiple_of((j // _LANES) * _LANES, _LANES)
            ivec = idx_v[pl.ds(chunk, _LANES)]
            a = jnp.max(jnp.where(riota == j % _LANES, ivec, 0))
            c0 = pl.multiple_of((a // _TILE) * _TILE, _TILE)
            lanes.append(a % _TILE)
            copies.append(pltpu.async_copy(
                pos_hbm.at[:, pl.ds(c0, _TILE)], wins[p], sems[p]))
        for p in range(jpw):
            copies[p].wait()
            cvec = jnp.full((_LANES,), lanes[p], jnp.int32)
            for t in range(steps):
                out_v[pl.ds(p * nrow + t * _LANES, _LANES)] = plsc.load_gather(
                    wins[p], [riota + t * _LANES, cvec])
        pltpu.sync_copy(out_v, out_hbm.at[pl.ds(wid * opw, opw)])

    out = gather_cols(pos2, idx32)
    return (out.reshape(n_idx, dim, batch)
            .transpose(2, 0, 1)
            .reshape(batch, n_idx * dim))
